# baseline (speedup 1.0000x reference)
import jax
import jax.numpy as jnp
from jax.experimental import pallas as pl


def _body(dig_ref, x_ref, enc_ref, dec_ref, heads_ref, rep_ref, recon_ref, z_ref):
    pre = jnp.dot(x_ref[...], enc_ref[...], preferred_element_type=jnp.float32)
    rep = jnp.tanh(pre)
    rep_ref[...] = rep
    rep16 = rep.astype(jnp.bfloat16)
    recon_ref[...] = jnp.dot(rep16, dec_ref[...], preferred_element_type=jnp.float32)
    dig = dig_ref[...]
    K = heads_ref.shape[0]
    acc = jnp.zeros(z_ref.shape, jnp.float32)
    for k in range(K):
        zk = jnp.dot(rep16, heads_ref[k], preferred_element_type=jnp.float32)
        acc = acc + jnp.where(dig == k, zk, 0.0)
    z_ref[...] = acc


def kernel(x_scaled, digits, W_enc, W_dec, heads):
    B, D_IN = x_scaled.shape
    REP = W_enc.shape[1]
    K, _, ZD = heads.shape
    BT = 512
    nb = B // BT
    dig2 = digits.reshape(B, 1)
    x16 = x_scaled.astype(jnp.bfloat16)
    enc16 = W_enc.astype(jnp.bfloat16)
    dec16 = W_dec.astype(jnp.bfloat16)
    heads16 = heads.astype(jnp.bfloat16)

    rep, recon, z = pl.pallas_call(
        _body,
        grid=(nb,),
        in_specs=[
            pl.BlockSpec((BT, 1), lambda i: (i, 0)),
            pl.BlockSpec((BT, D_IN), lambda i: (i, 0)),
            pl.BlockSpec((D_IN, REP), lambda i: (0, 0)),
            pl.BlockSpec((REP, D_IN), lambda i: (0, 0)),
            pl.BlockSpec((K, REP, ZD), lambda i: (0, 0, 0)),
        ],
        out_specs=[
            pl.BlockSpec((BT, REP), lambda i: (i, 0)),
            pl.BlockSpec((BT, D_IN), lambda i: (i, 0)),
            pl.BlockSpec((BT, ZD), lambda i: (i, 0)),
        ],
        out_shape=[
            jax.ShapeDtypeStruct((B, REP), jnp.float32),
            jax.ShapeDtypeStruct((B, D_IN), jnp.float32),
            jax.ShapeDtypeStruct((B, ZD), jnp.float32),
        ],
    )(dig2, x16, enc16, dec16, heads16)
    return rep, recon, z


# speedup vs baseline: 1.3008x; 1.3008x over previous
import jax
import jax.numpy as jnp
from jax.experimental import pallas as pl


def _body(dig_ref, x_ref, enc_ref, dec_ref, heads_ref, rep_ref, recon_ref, z_ref):
    pre = jnp.dot(x_ref[...].astype(jnp.bfloat16), enc_ref[...],
                  preferred_element_type=jnp.float32)
    rep = jnp.tanh(pre)
    rep_ref[...] = rep
    rep16 = rep.astype(jnp.bfloat16)
    recon_ref[...] = jnp.dot(rep16, dec_ref[...], preferred_element_type=jnp.float32)
    dig = dig_ref[...]
    K = heads_ref.shape[0]
    acc = jnp.zeros(z_ref.shape, jnp.float32)
    for k in range(K):
        zk = jnp.dot(rep16, heads_ref[k], preferred_element_type=jnp.float32)
        acc = acc + jnp.where(dig == k, zk, 0.0)
    z_ref[...] = acc


def kernel(x_scaled, digits, W_enc, W_dec, heads):
    B, D_IN = x_scaled.shape
    REP = W_enc.shape[1]
    K, _, ZD = heads.shape
    BT = 512
    nb = B // BT
    dig2 = digits.reshape(B, 1)
    enc16 = W_enc.astype(jnp.bfloat16)
    dec16 = W_dec.astype(jnp.bfloat16)
    heads16 = heads.astype(jnp.bfloat16)

    rep, recon, z = pl.pallas_call(
        _body,
        grid=(nb,),
        in_specs=[
            pl.BlockSpec((BT, 1), lambda i: (i, 0)),
            pl.BlockSpec((BT, D_IN), lambda i: (i, 0)),
            pl.BlockSpec((D_IN, REP), lambda i: (0, 0)),
            pl.BlockSpec((REP, D_IN), lambda i: (0, 0)),
            pl.BlockSpec((K, REP, ZD), lambda i: (0, 0, 0)),
        ],
        out_specs=[
            pl.BlockSpec((BT, REP), lambda i: (i, 0)),
            pl.BlockSpec((BT, D_IN), lambda i: (i, 0)),
            pl.BlockSpec((BT, ZD), lambda i: (i, 0)),
        ],
        out_shape=[
            jax.ShapeDtypeStruct((B, REP), jnp.float32),
            jax.ShapeDtypeStruct((B, D_IN), jnp.float32),
            jax.ShapeDtypeStruct((B, ZD), jnp.float32),
        ],
    )(dig2, x_scaled, enc16, dec16, heads16)
    return rep, recon, z


# speedup vs baseline: 1.5604x; 1.1996x over previous
import jax
import jax.numpy as jnp
from jax.experimental import pallas as pl


def _body(dig_ref, x_ref, enc_ref, dec_ref, headsw_ref, rep_ref, recon_ref, z_ref):
    ZD = z_ref.shape[1]
    K = headsw_ref.shape[1] // ZD
    rep = jnp.tanh(jnp.dot(x_ref[...], enc_ref[...], preferred_element_type=jnp.float32))
    rep_ref[...] = rep
    recon_ref[...] = jnp.dot(rep, dec_ref[...], preferred_element_type=jnp.float32)
    zw = jnp.dot(rep, headsw_ref[...], preferred_element_type=jnp.float32)
    dig = dig_ref[...]
    acc = zw[:, 0:ZD]
    for k in range(1, K):
        acc = jnp.where(dig == k, zw[:, k * ZD:(k + 1) * ZD], acc)
    z_ref[...] = acc


def kernel(x_scaled, digits, W_enc, W_dec, heads):
    B, D_IN = x_scaled.shape
    REP = W_enc.shape[1]
    K, _, ZD = heads.shape
    BT = 512
    nb = B // BT
    dig2 = digits.reshape(B, 1)
    heads_wide = heads.transpose(1, 0, 2).reshape(REP, K * ZD)

    rep, recon, z = pl.pallas_call(
        _body,
        grid=(nb,),
        in_specs=[
            pl.BlockSpec((BT, 1), lambda i: (i, 0)),
            pl.BlockSpec((BT, D_IN), lambda i: (i, 0)),
            pl.BlockSpec((D_IN, REP), lambda i: (0, 0)),
            pl.BlockSpec((REP, D_IN), lambda i: (0, 0)),
            pl.BlockSpec((REP, K * ZD), lambda i: (0, 0)),
        ],
        out_specs=[
            pl.BlockSpec((BT, REP), lambda i: (i, 0)),
            pl.BlockSpec((BT, D_IN), lambda i: (i, 0)),
            pl.BlockSpec((BT, ZD), lambda i: (i, 0)),
        ],
        out_shape=[
            jax.ShapeDtypeStruct((B, REP), jnp.float32),
            jax.ShapeDtypeStruct((B, D_IN), jnp.float32),
            jax.ShapeDtypeStruct((B, ZD), jnp.float32),
        ],
    )(dig2, x_scaled, W_enc, W_dec, heads_wide)
    return rep, recon, z


# speedup vs baseline: 1.6811x; 1.0774x over previous
import jax
import jax.numpy as jnp
from jax.experimental import pallas as pl


def _body(dig_ref, x_ref, enc_ref, dec_ref, headsw_ref, rep_ref, recon_ref, z_ref):
    ZD = z_ref.shape[1]
    K = headsw_ref.shape[1] // ZD
    rep = jnp.tanh(jnp.dot(x_ref[...], enc_ref[...], preferred_element_type=jnp.float32))
    rep_ref[...] = rep
    recon_ref[...] = jnp.dot(rep, dec_ref[...], preferred_element_type=jnp.float32)
    zw = jnp.dot(rep, headsw_ref[...], preferred_element_type=jnp.float32)
    dig = dig_ref[...]
    acc = zw[:, 0:ZD]
    for k in range(1, K):
        acc = jnp.where(dig == k, zw[:, k * ZD:(k + 1) * ZD], acc)
    z_ref[...] = acc


def kernel(x_scaled, digits, W_enc, W_dec, heads):
    B, D_IN = x_scaled.shape
    REP = W_enc.shape[1]
    K, _, ZD = heads.shape
    BT = 1024
    nb = B // BT
    dig2 = digits.reshape(B, 1)
    heads_wide = heads.transpose(1, 0, 2).reshape(REP, K * ZD)

    rep, recon, z = pl.pallas_call(
        _body,
        grid=(nb,),
        in_specs=[
            pl.BlockSpec((BT, 1), lambda i: (i, 0)),
            pl.BlockSpec((BT, D_IN), lambda i: (i, 0)),
            pl.BlockSpec((D_IN, REP), lambda i: (0, 0)),
            pl.BlockSpec((REP, D_IN), lambda i: (0, 0)),
            pl.BlockSpec((REP, K * ZD), lambda i: (0, 0)),
        ],
        out_specs=[
            pl.BlockSpec((BT, REP), lambda i: (i, 0)),
            pl.BlockSpec((BT, D_IN), lambda i: (i, 0)),
            pl.BlockSpec((BT, ZD), lambda i: (i, 0)),
        ],
        out_shape=[
            jax.ShapeDtypeStruct((B, REP), jnp.float32),
            jax.ShapeDtypeStruct((B, D_IN), jnp.float32),
            jax.ShapeDtypeStruct((B, ZD), jnp.float32),
        ],
    )(dig2, x_scaled, W_enc, W_dec, heads_wide)
    return rep, recon, z


# speedup vs baseline: 1.6902x; 1.0054x over previous
import jax
import jax.numpy as jnp
from jax.experimental import pallas as pl
from jax.experimental.pallas import tpu as pltpu


def _body(dig_ref, x_ref, enc_ref, dec_ref, headsw_ref, rep_ref, recon_ref, z_ref):
    ZD = z_ref.shape[1]
    K = headsw_ref.shape[1] // ZD
    rep = jnp.tanh(jnp.dot(x_ref[...], enc_ref[...], preferred_element_type=jnp.float32))
    rep_ref[...] = rep
    recon_ref[...] = jnp.dot(rep, dec_ref[...], preferred_element_type=jnp.float32)
    zw = jnp.dot(rep, headsw_ref[...], preferred_element_type=jnp.float32)
    dig = dig_ref[...]
    acc = zw[:, 0:ZD]
    for k in range(1, K):
        acc = jnp.where(dig == k, zw[:, k * ZD:(k + 1) * ZD], acc)
    z_ref[...] = acc


def kernel(x_scaled, digits, W_enc, W_dec, heads):
    B, D_IN = x_scaled.shape
    REP = W_enc.shape[1]
    K, _, ZD = heads.shape
    BT = 1024
    nb = B // BT
    dig2 = digits.reshape(B, 1)
    heads_wide = heads.transpose(1, 0, 2).reshape(REP, K * ZD)

    rep, recon, z = pl.pallas_call(
        _body,
        grid=(nb,),
        in_specs=[
            pl.BlockSpec((BT, 1), lambda i: (i, 0)),
            pl.BlockSpec((BT, D_IN), lambda i: (i, 0)),
            pl.BlockSpec((D_IN, REP), lambda i: (0, 0)),
            pl.BlockSpec((REP, D_IN), lambda i: (0, 0)),
            pl.BlockSpec((REP, K * ZD), lambda i: (0, 0)),
        ],
        out_specs=[
            pl.BlockSpec((BT, REP), lambda i: (i, 0)),
            pl.BlockSpec((BT, D_IN), lambda i: (i, 0)),
            pl.BlockSpec((BT, ZD), lambda i: (i, 0)),
        ],
        out_shape=[
            jax.ShapeDtypeStruct((B, REP), jnp.float32),
            jax.ShapeDtypeStruct((B, D_IN), jnp.float32),
            jax.ShapeDtypeStruct((B, ZD), jnp.float32),
        ],
        compiler_params=pltpu.CompilerParams(
            dimension_semantics=("arbitrary",),
            vmem_limit_bytes=100 * 1024 * 1024,
        ),
    )(dig2, x_scaled, W_enc, W_dec, heads_wide)
    return rep, recon, z
